# Initial kernel scaffold; baseline (speedup 1.0000x reference)
import jax
import jax.numpy as jnp
from jax.experimental import pallas as pl


def kernel(pos_f, pos_e, h_f, h_e, f2e_index, e2f_index, params):
    raise NotImplementedError("write your pallas kernel here")


# speedup vs baseline: 2.0462x; 2.0462x over previous
import functools

import jax
import jax.numpy as jnp
from jax import lax
from jax.experimental import pallas as pl
from jax.experimental.pallas import tpu as pltpu


def _edge_body(hf_ref, he_ref, pos_ref, w1a_ref, w1b_ref, w1d_ref, b1_ref,
               w2_ref, b2_ref, *rest, with_x):
    if with_x:
        wx1_ref, bx1_ref, wx2_ref, bx2_ref, m_ref, t_ref = rest
    else:
        (m_ref,) = rest
    hf = hf_ref[...]
    he = he_ref[...]
    pos = pos_ref[...].astype(jnp.float32)
    rel = pos[:, 4:7] - pos[:, 0:3]
    dist = jnp.sqrt(jnp.sum(rel * rel, axis=-1, keepdims=True))
    pre = (jnp.dot(hf, w1a_ref[...], preferred_element_type=jnp.float32)
           + jnp.dot(he, w1b_ref[...], preferred_element_type=jnp.float32)
           + dist * w1d_ref[...].astype(jnp.float32)
           + b1_ref[...].astype(jnp.float32))
    h = (pre * jax.nn.sigmoid(pre)).astype(jnp.bfloat16)
    m = jnp.dot(h, w2_ref[...], preferred_element_type=jnp.float32) + b2_ref[...].astype(jnp.float32)
    m_ref[...] = m.astype(m_ref.dtype)
    if with_x:
        px = (jnp.dot(m.astype(jnp.bfloat16), wx1_ref[...], preferred_element_type=jnp.float32)
              + bx1_ref[...].astype(jnp.float32))
        hx = (px * jax.nn.sigmoid(px)).astype(jnp.bfloat16)
        x = jnp.dot(hx, wx2_ref[...], preferred_element_type=jnp.float32) + bx2_ref[...].astype(jnp.float32)
        t = rel * x
        t_ref[...] = jnp.pad(t, ((0, 0), (0, 5))).astype(t_ref.dtype)


def _edge_mlp(hf_g, he_g, pos8, w1a, w1b, w1d, b1, w2, b2,
              xw1=None, xb1=None, xw2=None, xb2=None, blk=2000):
    R = hf_g.shape[0]
    if R % blk:
        blk = R
    with_x = xw1 is not None
    grid = (R // blk,)
    row = lambda i: (i, 0)
    full = lambda i: (0, 0)
    in_specs = [
        pl.BlockSpec((blk, 128), row),
        pl.BlockSpec((blk, 128), row),
        pl.BlockSpec((blk, 8), row),
        pl.BlockSpec((128, 256), full),
        pl.BlockSpec((128, 256), full),
        pl.BlockSpec((1, 256), full),
        pl.BlockSpec((1, 256), full),
        pl.BlockSpec((256, 128), full),
        pl.BlockSpec((1, 128), full),
    ]
    out_shape = [jax.ShapeDtypeStruct((R, 128), jnp.bfloat16)]
    out_specs = [pl.BlockSpec((blk, 128), row)]
    args = [hf_g, he_g, pos8, w1a, w1b, w1d, b1, w2, b2]
    if with_x:
        in_specs += [
            pl.BlockSpec((128, 256), full),
            pl.BlockSpec((1, 256), full),
            pl.BlockSpec((256, 1), full),
            pl.BlockSpec((1, 1), full),
        ]
        args += [xw1, xb1, xw2, xb2]
        out_shape.append(jax.ShapeDtypeStruct((R, 8), jnp.float32))
        out_specs.append(pl.BlockSpec((blk, 8), row))
    res = pl.pallas_call(
        functools.partial(_edge_body, with_x=with_x),
        grid=grid,
        in_specs=in_specs,
        out_specs=out_specs,
        out_shape=out_shape,
    )(*args)
    return res if with_x else res[0]


def _node_body(h_ref, aggr_ref, w1a_ref, w1b_ref, b1_ref, w2_ref, b2_ref, out_ref):
    h = h_ref[...]
    a = aggr_ref[...]
    pre = (jnp.dot(h.astype(jnp.bfloat16), w1a_ref[...], preferred_element_type=jnp.float32)
           + jnp.dot(a.astype(jnp.bfloat16), w1b_ref[...], preferred_element_type=jnp.float32)
           + b1_ref[...].astype(jnp.float32))
    hh = (pre * jax.nn.sigmoid(pre)).astype(jnp.bfloat16)
    d = jnp.dot(hh, w2_ref[...], preferred_element_type=jnp.float32) + b2_ref[...].astype(jnp.float32)
    out_ref[...] = h.astype(jnp.float32) + d


def _node_mlp(h, aggr, w1a, w1b, b1, w2, b2, blk=2000):
    R = h.shape[0]
    if R % blk:
        blk = R
    grid = (R // blk,)
    row = lambda i: (i, 0)
    full = lambda i: (0, 0)
    return pl.pallas_call(
        _node_body,
        grid=grid,
        in_specs=[
            pl.BlockSpec((blk, 128), row),
            pl.BlockSpec((blk, 128), row),
            pl.BlockSpec((128, 256), full),
            pl.BlockSpec((128, 256), full),
            pl.BlockSpec((1, 256), full),
            pl.BlockSpec((256, 128), full),
            pl.BlockSpec((1, 128), full),
        ],
        out_specs=pl.BlockSpec((blk, 128), row),
        out_shape=jax.ShapeDtypeStruct((R, 128), jnp.float32),
    )(h, aggr, w1a, w1b, b1, w2, b2)


def kernel(pos_f, pos_e, h_f, h_e, f2e_index, e2f_index, params):
    B, NF, _ = h_f.shape
    NE = h_e.shape[1]
    M = f2e_index.shape[2]
    D = 128

    (W1_f2e, b1_f2e), (W2_f2e, b2_f2e) = params['f2e']
    (W1_e2f, b1_e2f), (W2_e2f, b2_e2f) = params['e2f']
    (Wx1, bx1), (Wx2, bx2) = params['x']
    (W1_hf, b1_hf), (W2_hf, b2_hf) = params['h_f']
    (W1_he, b1_he), (W2_he, b2_he) = params['h_e']

    bf = jnp.bfloat16
    cvt = lambda x: x.astype(bf)
    r2 = lambda v: v.reshape(1, -1)

    hf_flat = h_f.reshape(B * NF, D)
    he_flat = h_e.reshape(B * NE, D)
    pf_flat = pos_f.reshape(B * NF, 3)
    pe_flat = pos_e.reshape(B * NE, 3)
    hf_bf = cvt(hf_flat)
    he_bf = cvt(he_flat)

    off_f = (jnp.arange(B, dtype=jnp.int32) * NF)[:, None]
    off_e = (jnp.arange(B, dtype=jnp.int32) * NE)[:, None]

    idx_f = (f2e_index[:, 0, :] + off_f).reshape(-1)
    idx_e = (f2e_index[:, 1, :] + off_e).reshape(-1)
    hf_g = jnp.take(hf_bf, idx_f, axis=0)
    he_g = jnp.take(he_bf, idx_e, axis=0)
    pf_g = jnp.take(pf_flat, idx_f, axis=0)
    pe_g = jnp.take(pe_flat, idx_e, axis=0)
    pos8 = jnp.concatenate(
        [pf_g, jnp.zeros((B * M, 1), jnp.float32), pe_g, jnp.zeros((B * M, 1), jnp.float32)], axis=-1)

    m_f2e, trans8 = _edge_mlp(
        hf_g, he_g, pos8,
        cvt(W1_f2e[:D]), cvt(W1_f2e[D:2 * D]), cvt(r2(W1_f2e[2 * D])), cvt(r2(b1_f2e)),
        cvt(W2_f2e), cvt(r2(b2_f2e)),
        cvt(Wx1), cvt(r2(bx1)), cvt(Wx2), cvt(r2(bx2)))

    aggr_f2e = jnp.zeros((B * NE, D), jnp.float32).at[idx_e].add(m_f2e.astype(jnp.float32))
    pos_acc = jnp.zeros((B * NE, 8), jnp.float32).at[idx_e].add(trans8)
    pos_e_next = pos_e + pos_acc[:, :3].reshape(B, NE, 3)

    h_e_next = _node_mlp(
        he_flat, aggr_f2e,
        cvt(W1_he[:D]), cvt(W1_he[D:]), cvt(r2(b1_he)), cvt(W2_he), cvt(r2(b2_he)),
    ).reshape(B, NE, D)

    idx_er = (e2f_index[:, 0, :] + off_e).reshape(-1)
    idx_fr = (e2f_index[:, 1, :] + off_f).reshape(-1)
    he_gr = jnp.take(he_bf, idx_er, axis=0)
    hf_gr = jnp.take(hf_bf, idx_fr, axis=0)
    pe_gr = jnp.take(pe_flat, idx_er, axis=0)
    pf_gr = jnp.take(pf_flat, idx_fr, axis=0)
    pos8r = jnp.concatenate(
        [pe_gr, jnp.zeros((B * M, 1), jnp.float32), pf_gr, jnp.zeros((B * M, 1), jnp.float32)], axis=-1)

    m_e2f = _edge_mlp(
        he_gr, hf_gr, pos8r,
        cvt(W1_e2f[:D]), cvt(W1_e2f[D:2 * D]), cvt(r2(W1_e2f[2 * D])), cvt(r2(b1_e2f)),
        cvt(W2_e2f), cvt(r2(b2_e2f)))

    aggr_e2f = jnp.zeros((B * NF, D), jnp.float32).at[idx_fr].add(m_e2f.astype(jnp.float32))

    h_f_next = _node_mlp(
        hf_flat, aggr_e2f,
        cvt(W1_hf[:D]), cvt(W1_hf[D:]), cvt(r2(b1_hf)), cvt(W2_hf), cvt(r2(b2_hf)),
    ).reshape(B, NF, D)

    return (pos_f, pos_e_next, h_f_next, h_e_next)


# speedup vs baseline: 5.9172x; 2.8919x over previous
import functools

import jax
import jax.numpy as jnp
from jax import lax
from jax.experimental import pallas as pl
from jax.experimental.pallas import tpu as pltpu
from jax.experimental.pallas import tpu_sc as plsc

_SC_NC = 2
_SC_NW = 32


def _sc_gather4(idx_f, idx_e, tab_f, tab_e, ptab_f, ptab_e, ch=192):
    npad = idx_f.shape[0]
    per_w = npad // _SC_NW
    while per_w % ch:
        ch //= 2
    n_ch = per_w // ch
    fdt = tab_f.dtype
    mesh = plsc.VectorSubcoreMesh(core_axis_name="c", subcore_axis_name="s",
                                  num_cores=_SC_NC, num_subcores=_SC_NW // _SC_NC)

    def body(idxf_hbm, idxe_hbm, tf_hbm, te_hbm, pf_hbm, pe_hbm,
             of_hbm, oe_hbm, opf_hbm, ope_hbm,
             idxf_v, idxe_v, bf, be, bpf, bpe, sem):
        wid = lax.axis_index("s") * _SC_NC + lax.axis_index("c")
        base = wid * per_w
        pltpu.sync_copy(idxf_hbm.at[pl.ds(base, per_w)], idxf_v)
        pltpu.sync_copy(idxe_hbm.at[pl.ds(base, per_w)], idxe_v)

        def step(i, carry):
            off = i * ch
            d1 = pltpu.async_copy(tf_hbm.at[idxf_v.at[pl.ds(off, ch)]], bf, sem)
            d2 = pltpu.async_copy(te_hbm.at[idxe_v.at[pl.ds(off, ch)]], be, sem)
            d3 = pltpu.async_copy(pf_hbm.at[idxf_v.at[pl.ds(off, ch)]], bpf, sem)
            d4 = pltpu.async_copy(pe_hbm.at[idxe_v.at[pl.ds(off, ch)]], bpe, sem)
            d1.wait()
            d2.wait()
            d3.wait()
            d4.wait()
            pltpu.sync_copy(bf, of_hbm.at[pl.ds(base + off, ch)])
            pltpu.sync_copy(be, oe_hbm.at[pl.ds(base + off, ch)])
            pltpu.sync_copy(bpf, opf_hbm.at[pl.ds(base + off, ch)])
            pltpu.sync_copy(bpe, ope_hbm.at[pl.ds(base + off, ch)])
            return carry

        lax.fori_loop(0, n_ch, step, 0)

    return pl.kernel(
        body,
        out_type=[
            jax.ShapeDtypeStruct((npad, 128), fdt),
            jax.ShapeDtypeStruct((npad, 128), fdt),
            jax.ShapeDtypeStruct((npad, 128), jnp.float32),
            jax.ShapeDtypeStruct((npad, 128), jnp.float32),
        ],
        mesh=mesh,
        scratch_types=[
            pltpu.VMEM((per_w,), jnp.int32),
            pltpu.VMEM((per_w,), jnp.int32),
            pltpu.VMEM((ch, 128), fdt),
            pltpu.VMEM((ch, 128), fdt),
            pltpu.VMEM((ch, 128), jnp.float32),
            pltpu.VMEM((ch, 128), jnp.float32),
            pltpu.SemaphoreType.DMA,
        ],
    )(idx_f, idx_e, tab_f, tab_e, ptab_f, ptab_e)


def _edge_body(hf_ref, he_ref, pf_ref, pe_ref, w1a_ref, w1b_ref, w1d_ref, b1_ref,
               w2_ref, b2_ref, *rest, with_x):
    if with_x:
        wx1_ref, bx1_ref, wx2_ref, bx2_ref, m_ref, t_ref = rest
    else:
        (m_ref,) = rest
    hf = hf_ref[...].astype(jnp.bfloat16)
    he = he_ref[...].astype(jnp.bfloat16)
    pf = pf_ref[...]
    pe = pe_ref[...]
    rel = pe[:, 0:3] - pf[:, 0:3]
    dist = jnp.sqrt(jnp.sum(rel * rel, axis=-1, keepdims=True))
    pre = (jnp.dot(hf, w1a_ref[...], preferred_element_type=jnp.float32)
           + jnp.dot(he, w1b_ref[...], preferred_element_type=jnp.float32)
           + dist * w1d_ref[...].astype(jnp.float32)
           + b1_ref[...].astype(jnp.float32))
    h = (pre * jax.nn.sigmoid(pre)).astype(jnp.bfloat16)
    m = jnp.dot(h, w2_ref[...], preferred_element_type=jnp.float32) + b2_ref[...].astype(jnp.float32)
    m_ref[...] = m.astype(m_ref.dtype)
    if with_x:
        px = (jnp.dot(m.astype(jnp.bfloat16), wx1_ref[...], preferred_element_type=jnp.float32)
              + bx1_ref[...].astype(jnp.float32))
        hx = (px * jax.nn.sigmoid(px)).astype(jnp.bfloat16)
        x = jnp.dot(hx, wx2_ref[...], preferred_element_type=jnp.float32) + bx2_ref[...].astype(jnp.float32)
        t = rel * x
        t_ref[...] = jnp.pad(t, ((0, 0), (0, 13))).astype(t_ref.dtype)


def _edge_mlp(hf_g, he_g, pf_g, pe_g, w1a, w1b, w1d, b1, w2, b2,
              xw1=None, xb1=None, xw2=None, xb2=None, blk=2048):
    R = hf_g.shape[0]
    if R % blk:
        blk = R
    with_x = xw1 is not None
    grid = (R // blk,)
    row = lambda i: (i, 0)
    full = lambda i: (0, 0)
    in_specs = [
        pl.BlockSpec((blk, 128), row),
        pl.BlockSpec((blk, 128), row),
        pl.BlockSpec((blk, 128), row),
        pl.BlockSpec((blk, 128), row),
        pl.BlockSpec((128, 256), full),
        pl.BlockSpec((128, 256), full),
        pl.BlockSpec((1, 256), full),
        pl.BlockSpec((1, 256), full),
        pl.BlockSpec((256, 128), full),
        pl.BlockSpec((1, 128), full),
    ]
    out_shape = [jax.ShapeDtypeStruct((R, 128), jnp.float32)]
    out_specs = [pl.BlockSpec((blk, 128), row)]
    args = [hf_g, he_g, pf_g, pe_g, w1a, w1b, w1d, b1, w2, b2]
    if with_x:
        in_specs += [
            pl.BlockSpec((128, 256), full),
            pl.BlockSpec((1, 256), full),
            pl.BlockSpec((256, 1), full),
            pl.BlockSpec((1, 1), full),
        ]
        args += [xw1, xb1, xw2, xb2]
        out_shape.append(jax.ShapeDtypeStruct((R, 16), jnp.float32))
        out_specs.append(pl.BlockSpec((blk, 16), row))
    res = pl.pallas_call(
        functools.partial(_edge_body, with_x=with_x),
        grid=grid,
        in_specs=in_specs,
        out_specs=out_specs,
        out_shape=out_shape,
    )(*args)
    return res if with_x else res[0]


def _node_body(h_ref, aggr_ref, w1a_ref, w1b_ref, b1_ref, w2_ref, b2_ref, out_ref):
    h = h_ref[...]
    a = aggr_ref[...]
    pre = (jnp.dot(h.astype(jnp.bfloat16), w1a_ref[...], preferred_element_type=jnp.float32)
           + jnp.dot(a.astype(jnp.bfloat16), w1b_ref[...], preferred_element_type=jnp.float32)
           + b1_ref[...].astype(jnp.float32))
    hh = (pre * jax.nn.sigmoid(pre)).astype(jnp.bfloat16)
    d = jnp.dot(hh, w2_ref[...], preferred_element_type=jnp.float32) + b2_ref[...].astype(jnp.float32)
    out_ref[...] = h.astype(jnp.float32) + d


def _node_mlp(h, aggr, w1a, w1b, b1, w2, b2, blk=2000):
    R = h.shape[0]
    if R % blk:
        blk = R
    grid = (R // blk,)
    row = lambda i: (i, 0)
    full = lambda i: (0, 0)
    return pl.pallas_call(
        _node_body,
        grid=grid,
        in_specs=[
            pl.BlockSpec((blk, 128), row),
            pl.BlockSpec((blk, 128), row),
            pl.BlockSpec((128, 256), full),
            pl.BlockSpec((128, 256), full),
            pl.BlockSpec((1, 256), full),
            pl.BlockSpec((256, 128), full),
            pl.BlockSpec((1, 128), full),
        ],
        out_specs=pl.BlockSpec((blk, 128), row),
        out_shape=jax.ShapeDtypeStruct((R, 128), jnp.float32),
    )(h, aggr, w1a, w1b, b1, w2, b2)


def kernel(pos_f, pos_e, h_f, h_e, f2e_index, e2f_index, params):
    B, NF, _ = h_f.shape
    NE = h_e.shape[1]
    M = f2e_index.shape[2]
    D = 128

    (W1_f2e, b1_f2e), (W2_f2e, b2_f2e) = params['f2e']
    (W1_e2f, b1_e2f), (W2_e2f, b2_e2f) = params['e2f']
    (Wx1, bx1), (Wx2, bx2) = params['x']
    (W1_hf, b1_hf), (W2_hf, b2_hf) = params['h_f']
    (W1_he, b1_he), (W2_he, b2_he) = params['h_e']

    bf = jnp.bfloat16
    cvt = lambda x: x.astype(bf)
    r2 = lambda v: v.reshape(1, -1)

    PADB = ((M + 2047) // 2048) * 2048
    NPAD = B * PADB

    hf_flat = h_f.reshape(B * NF, D)
    he_flat = h_e.reshape(B * NE, D)
    ptab_f = jnp.pad(pos_f.reshape(B * NF, 3), ((0, 0), (0, 125)))
    ptab_e = jnp.pad(pos_e.reshape(B * NE, 3), ((0, 0), (0, 125)))

    off_f = (jnp.arange(B, dtype=jnp.int32) * NF)[:, None]
    off_e = (jnp.arange(B, dtype=jnp.int32) * NE)[:, None]
    padidx = lambda a, off: (jnp.pad(a, ((0, 0), (0, PADB - M))) + off).reshape(-1)

    idx_f = padidx(f2e_index[:, 0, :], off_f)
    idx_e = padidx(f2e_index[:, 1, :], off_e)
    hf_g, he_g, pf_g, pe_g = _sc_gather4(idx_f, idx_e, hf_flat, he_flat, ptab_f, ptab_e)

    m_f2e, trans = _edge_mlp(
        hf_g, he_g, pf_g, pe_g,
        cvt(W1_f2e[:D]), cvt(W1_f2e[D:2 * D]), cvt(r2(W1_f2e[2 * D])), cvt(r2(b1_f2e)),
        cvt(W2_f2e), cvt(r2(b2_f2e)),
        cvt(Wx1), cvt(r2(bx1)), cvt(Wx2), cvt(r2(bx2)))

    midx_e = idx_e.reshape(B, PADB)[:, :M].reshape(-1)
    m_f2e_r = m_f2e.reshape(B, PADB, D)[:, :M].reshape(B * M, D)
    trans_r = trans.reshape(B, PADB, 16)[:, :M].reshape(B * M, 16)
    aggr_f2e = jnp.zeros((B * NE, D), jnp.float32).at[midx_e].add(m_f2e_r)
    pos_acc = jnp.zeros((B * NE, 16), jnp.float32).at[midx_e].add(trans_r)
    pos_e_next = pos_e + pos_acc[:, :3].reshape(B, NE, 3)

    h_e_next = _node_mlp(
        he_flat, aggr_f2e,
        cvt(W1_he[:D]), cvt(W1_he[D:]), cvt(r2(b1_he)), cvt(W2_he), cvt(r2(b2_he)),
    ).reshape(B, NE, D)

    idx_er = padidx(e2f_index[:, 0, :], off_e)
    idx_fr = padidx(e2f_index[:, 1, :], off_f)
    he_gr, hf_gr, pe_gr, pf_gr = _sc_gather4(idx_er, idx_fr, he_flat, hf_flat, ptab_e, ptab_f)

    m_e2f = _edge_mlp(
        he_gr, hf_gr, pe_gr, pf_gr,
        cvt(W1_e2f[:D]), cvt(W1_e2f[D:2 * D]), cvt(r2(W1_e2f[2 * D])), cvt(r2(b1_e2f)),
        cvt(W2_e2f), cvt(r2(b2_e2f)))

    midx_f = idx_fr.reshape(B, PADB)[:, :M].reshape(-1)
    m_e2f_r = m_e2f.reshape(B, PADB, D)[:, :M].reshape(B * M, D)
    aggr_e2f = jnp.zeros((B * NF, D), jnp.float32).at[midx_f].add(m_e2f_r)

    h_f_next = _node_mlp(
        hf_flat, aggr_e2f,
        cvt(W1_hf[:D]), cvt(W1_hf[D:]), cvt(r2(b1_hf)), cvt(W2_hf), cvt(r2(b2_hf)),
    ).reshape(B, NF, D)

    return (pos_f, pos_e_next, h_f_next, h_e_next)
